# Initial kernel scaffold; baseline (speedup 1.0000x reference)
import jax
import jax.numpy as jnp
from jax.experimental import pallas as pl


def kernel(X_sp, P_sp, W0, g0, b0, a0, W1, g1, b1, a1, Wp1, bp1, Wp2, bp2):
    raise NotImplementedError("write your pallas kernel here")


# speedup vs baseline: 3.8629x; 3.8629x over previous
import functools

import jax
import jax.numpy as jnp
from jax import lax
from jax.experimental import pallas as pl
from jax.experimental.pallas import tpu as pltpu
from jax.experimental.pallas import tpu_sc as plsc

N = 4096
DIM = 128
KNN = 10
KP = 16
ETA = 0.5
DELTA = 1.0
RB = 256
NBLK = N // RB

NC = 2
NS = 16
NW = NC * NS
RPW = N // NW
CROWS = 8
NCHUNK = RPW // CROWS
CEDGE = CROWS * KP


def _max_body(x_ref, p_ref, ms_ref, mp_ref, acc_ref):
    b = pl.program_id(0)
    x = x_ref[...]
    p = p_ref[...]
    xr = x_ref[pl.ds(b * RB, RB), :]
    pr = p_ref[pl.ds(b * RB, RB), :]

    def d2max(ar, a):
        g = lax.dot_general(ar.astype(jnp.bfloat16), a.astype(jnp.bfloat16),
                            (((1,), (1,)), ((), ())),
                            preferred_element_type=jnp.float32)
        a2r = jnp.sum(ar * ar, axis=1, keepdims=True)
        ones = jnp.ones((1, DIM), jnp.float32)
        a2c = lax.dot_general(ones, a * a, (((1,), (1,)), ((), ())),
                              precision=lax.Precision.HIGHEST,
                              preferred_element_type=jnp.float32)
        return jnp.max(a2r + a2c - 2.0 * g)

    ms = d2max(xr, x)
    mp = d2max(pr, p)

    @pl.when(b == 0)
    def _():
        acc_ref[0] = ms
        acc_ref[1] = mp

    @pl.when(b > 0)
    def _():
        acc_ref[0] = jnp.maximum(acc_ref[0], ms)
        acc_ref[1] = jnp.maximum(acc_ref[1], mp)

    @pl.when(b == NBLK - 1)
    def _():
        ms_ref[0, 0] = jnp.sqrt(jnp.clip(acc_ref[0], 1e-12, None))
        mp_ref[0, 0] = jnp.sqrt(jnp.clip(acc_ref[1], 1e-12, None))


def _pass_max(X, Pp):
    return pl.pallas_call(
        _max_body,
        grid=(NBLK,),
        in_specs=[
            pl.BlockSpec((N, DIM), lambda b: (0, 0)),
            pl.BlockSpec((N, DIM), lambda b: (0, 0)),
        ],
        out_specs=[
            pl.BlockSpec(memory_space=pltpu.SMEM),
            pl.BlockSpec(memory_space=pltpu.SMEM),
        ],
        out_shape=[
            jax.ShapeDtypeStruct((1, 1), jnp.float32),
            jax.ShapeDtypeStruct((1, 1), jnp.float32),
        ],
        scratch_shapes=[pltpu.SMEM((2,), jnp.float32)],
    )(X, Pp)


def _topk_body(x_ref, p_ref, ms_ref, mp_ref, idx_ref, w_ref):
    b = pl.program_id(0)
    x = x_ref[...]
    p = p_ref[...]
    xr = x_ref[pl.ds(b * RB, RB), :]
    pr = p_ref[pl.ds(b * RB, RB), :]
    ms = ms_ref[0, 0]
    mp = mp_ref[0, 0]

    def d2(ar, a):
        g = lax.dot_general(ar.astype(jnp.bfloat16), a.astype(jnp.bfloat16),
                            (((1,), (1,)), ((), ())),
                            preferred_element_type=jnp.float32)
        a2r = jnp.sum(ar * ar, axis=1, keepdims=True)
        ones = jnp.ones((1, DIM), jnp.float32)
        a2c = lax.dot_general(ones, a * a, (((1,), (1,)), ((), ())),
                              precision=lax.Precision.HIGHEST,
                              preferred_element_type=jnp.float32)
        return jnp.clip(a2r + a2c - 2.0 * g, 1e-12, None)

    D = (ETA * (jnp.sqrt(d2(pr, p)) / (mp + 1e-8))
         + (1.0 - ETA) * (jnp.sqrt(d2(xr, x)) / (ms + 1e-8)))

    jglob = lax.broadcasted_iota(jnp.int32, (RB, N), 1)
    ig = lax.broadcasted_iota(jnp.int32, (RB, 1), 0) + b * RB
    Dm = jnp.where(jglob == ig, jnp.inf, D)

    for k in range(KNN):
        m = jnp.min(Dm, axis=1, keepdims=True)
        sel = jnp.min(jnp.where(Dm == m, jglob, N), axis=1,
                      keepdims=True)
        idx_ref[:, k:k + 1] = sel
        w_ref[:, k:k + 1] = jnp.exp(-(m * m) / (DELTA * DELTA + 1e-8))
        Dm = jnp.where(jglob == sel, jnp.inf, Dm)

    for k in range(KNN, KP):
        idx_ref[:, k:k + 1] = ig
        w_ref[:, k:k + 1] = jnp.zeros((RB, 1), jnp.float32)


def _pass_topk(X, Pp, ms, mp):
    return pl.pallas_call(
        _topk_body,
        grid=(NBLK,),
        in_specs=[
            pl.BlockSpec((N, DIM), lambda b: (0, 0)),
            pl.BlockSpec((N, DIM), lambda b: (0, 0)),
            pl.BlockSpec(memory_space=pltpu.SMEM),
            pl.BlockSpec(memory_space=pltpu.SMEM),
        ],
        out_specs=[
            pl.BlockSpec((RB, KP), lambda b: (b, 0)),
            pl.BlockSpec((RB, KP), lambda b: (b, 0)),
        ],
        out_shape=[
            jax.ShapeDtypeStruct((N, KP), jnp.int32),
            jax.ShapeDtypeStruct((N, KP), jnp.float32),
        ],
    )(X, Pp, ms, mp)


def _sc_deg_body(idx2d, idxf, wf, wgf, idx_v, w_v, wg_v, nbr, sem):
    cid = lax.axis_index("c")
    sid = lax.axis_index("s")
    wid = sid * NC + cid
    lane = lax.iota(jnp.int32, 16)
    rots = [((lane + sh) & 15) for sh in (8, 4, 2, 1)]

    def chunk_body(c, _):
        ebase = wid * (RPW * KP) + c * CEDGE
        pltpu.sync_copy(idxf.at[pl.ds(ebase, CEDGE)], idx_v)
        pltpu.sync_copy(wf.at[pl.ds(ebase, CEDGE)], w_v)
        pltpu.async_copy(idx2d.at[idx_v], nbr, sem).wait()

        def grp_body(g, _):
            i_row = wid * RPW + c * CROWS + g
            e0 = g * KP
            wvec = w_v[pl.ds(e0, 16)]
            macc = jnp.zeros((16,), jnp.int32)
            for t in range(KP):
                row = nbr[e0 + t, pl.ds(0, 16)]
                acc = jnp.where(row == i_row, 1, 0)
                for rot in rots:
                    acc = acc | acc.at[rot].get(mode="promise_in_bounds")
                macc = jnp.where(lane == t, acc, macc)
            wg_v[pl.ds(e0, 16)] = jnp.where(macc > 0, 0.0, wvec)
            return 0

        lax.fori_loop(0, CROWS, grp_body, 0)
        pltpu.sync_copy(wg_v, wgf.at[pl.ds(ebase, CEDGE)])
        return 0

    lax.fori_loop(0, NCHUNK, chunk_body, 0)


def _pass_sc_deg(idx2d, idxf, wf):
    mesh = plsc.VectorSubcoreMesh(core_axis_name="c", subcore_axis_name="s",
                                  num_cores=NC)
    f = functools.partial(
        pl.kernel,
        mesh=mesh,
        out_type=jax.ShapeDtypeStruct((N * KP,), jnp.float32),
        scratch_types=[
            pltpu.VMEM((CEDGE,), jnp.int32),
            pltpu.VMEM((CEDGE,), jnp.float32),
            pltpu.VMEM((CEDGE,), jnp.float32),
            pltpu.VMEM((CEDGE, DIM), jnp.int32),
            pltpu.SemaphoreType.DMA,
        ],
    )(_sc_deg_body)
    return f(idx2d, idxf, wf)


def _sc_agg_body(hs, idxf, wf, wgf, outp, shared, idx_v, w_v, wg_v, gath,
                 valsb, own, gacc, oidx, sem):
    cid = lax.axis_index("c")
    sid = lax.axis_index("s")
    wid = sid * NC + cid
    lane = lax.iota(jnp.int32, 16)
    nvec = DIM // 16

    def zb(i, _):
        gath[i // nvec, pl.ds((i % nvec) * 16, 16)] = jnp.zeros(
            (16,), jnp.float32)
        return 0

    lax.fori_loop(0, CEDGE * nvec, zb, 0)
    pltpu.sync_copy(gath, shared.at[pl.ds(sid * 256, CEDGE)])
    pltpu.sync_copy(gath, shared.at[pl.ds(sid * 256 + CEDGE, CEDGE)])
    plsc.subcore_barrier()

    def gz(i, _):
        gacc[CROWS + i // nvec, pl.ds((i % nvec) * 16, 16)] = jnp.zeros(
            (16,), jnp.float32)
        return 0

    lax.fori_loop(0, (KP - CROWS) * nvec, gz, 0)

    def chunk_body(c, _):
        rowbase = wid * RPW + c * CROWS
        ebase = rowbase * KP
        pltpu.sync_copy(idxf.at[pl.ds(ebase, CEDGE)], idx_v)
        pltpu.sync_copy(wf.at[pl.ds(ebase, CEDGE)], w_v)
        pltpu.sync_copy(wgf.at[pl.ds(ebase, CEDGE)], wg_v)
        pltpu.async_copy(hs.at[idx_v], gath, sem).wait()
        pltpu.sync_copy(hs.at[pl.ds(rowbase, CROWS)], own)

        def row_body(r, _):
            wrow = w_v[pl.ds(r * KP, KP)]
            wgrow = wg_v[pl.ds(r * KP, KP)]
            ownd = [own[r, pl.ds(d * 16, 16)] for d in range(nvec)]
            accs = [jnp.zeros((16,), jnp.float32) for _ in range(nvec)]
            for t in range(KP):
                wsc = wrow[t]
                wgsc = wgrow[t]
                e = r * KP + t
                for d in range(nvec):
                    valsb[e, pl.ds(d * 16, 16)] = wsc * ownd[d]
                    accs[d] = accs[d] + wgsc * gath[e, pl.ds(d * 16, 16)]
            for d in range(nvec):
                gacc[r, pl.ds(d * 16, 16)] = accs[d]
            return 0

        lax.fori_loop(0, CROWS, row_body, 0)

        oidx[...] = rowbase + (lane & (CROWS - 1))
        pltpu.sync_copy(valsb, shared.at[idx_v], add=True)
        pltpu.sync_copy(gacc, shared.at[oidx], add=True)
        return 0

    lax.fori_loop(0, NCHUNK, chunk_body, 0)
    plsc.subcore_barrier()
    base = sid * 256
    pltpu.sync_copy(shared.at[pl.ds(base, 256)],
                    outp.at[pl.ds(cid * N + base, 256)])


def _pass_sc_agg(hs, idxf, wf, wgf):
    mesh = plsc.VectorSubcoreMesh(core_axis_name="c", subcore_axis_name="s",
                                  num_cores=NC)
    f = functools.partial(
        pl.kernel,
        mesh=mesh,
        out_type=jax.ShapeDtypeStruct((NC * N, DIM), jnp.float32),
        scratch_types=[
            pltpu.VMEM_SHARED((N, DIM), jnp.float32),
            pltpu.VMEM((CEDGE,), jnp.int32),
            pltpu.VMEM((CEDGE,), jnp.float32),
            pltpu.VMEM((CEDGE,), jnp.float32),
            pltpu.VMEM((CEDGE, DIM), jnp.float32),
            pltpu.VMEM((CEDGE, DIM), jnp.float32),
            pltpu.VMEM((CROWS, DIM), jnp.float32),
            pltpu.VMEM((KP, DIM), jnp.float32),
            pltpu.VMEM((16,), jnp.int32),
            pltpu.SemaphoreType.DMA,
        ],
    )(_sc_agg_body)
    return f(hs, idxf, wf, wgf)


def _pre1_body(x_ref, w_ref, degs_ref, hs_ref, dinv_ref):
    s = degs_ref[pl.ds(0, N), 0:1] + degs_ref[pl.ds(N, N), 0:1]
    deg = jnp.clip(1.0 + s, 1.0, None)
    dinv = deg ** -0.5
    dinv_ref[...] = dinv
    h = lax.dot_general(x_ref[...], w_ref[...], (((1,), (1,)), ((), ())),
                        preferred_element_type=jnp.float32)
    hs_ref[...] = h * dinv


def _pass_pre1(X, W0, degs):
    return pl.pallas_call(
        _pre1_body,
        out_shape=[
            jax.ShapeDtypeStruct((N, DIM), jnp.float32),
            jax.ShapeDtypeStruct((N, 1), jnp.float32),
        ],
    )(X, W0, degs)


def _pre2_body(x_ref, w_ref, dinv_ref, hs_ref):
    h = lax.dot_general(x_ref[...], w_ref[...], (((1,), (1,)), ((), ())),
                        preferred_element_type=jnp.float32)
    hs_ref[...] = h * dinv_ref[...]


def _pass_pre2(H, W1, dinv):
    return pl.pallas_call(
        _pre2_body,
        out_shape=jax.ShapeDtypeStruct((N, DIM), jnp.float32),
    )(H, W1, dinv)


def _post_math(outp, hs, dinv, g, b, a):
    agg = dinv * (outp[pl.ds(0, N), :] + outp[pl.ds(N, N), :] + hs)
    mu = jnp.mean(agg, axis=0)
    var = jnp.mean((agg - mu) ** 2, axis=0)
    h = (agg - mu) / jnp.sqrt(var + 1e-5) * g + b
    return jnp.where(h > 0, h, a * h)


def _post1_body(outp_ref, hs_ref, dinv_ref, g_ref, b_ref, a_ref, out_ref):
    out_ref[...] = _post_math(outp_ref, hs_ref[...], dinv_ref[...],
                              g_ref[...], b_ref[...], a_ref[...])


def _pass_post1(outp, hs, dinv, g, b, a):
    return pl.pallas_call(
        _post1_body,
        out_shape=jax.ShapeDtypeStruct((N, DIM), jnp.float32),
    )(outp, hs, dinv, g, b, a)


def _post2_body(outp_ref, hs_ref, dinv_ref, g_ref, b_ref, a_ref,
                wp1_ref, bp1_ref, wp2_ref, bp2_ref, out_ref):
    h = _post_math(outp_ref, hs_ref[...], dinv_ref[...],
                   g_ref[...], b_ref[...], a_ref[...])
    z = jnp.maximum(
        lax.dot_general(h, wp1_ref[...], (((1,), (1,)), ((), ())),
                        preferred_element_type=jnp.float32) + bp1_ref[...],
        0.0)
    z = lax.dot_general(z, wp2_ref[...], (((1,), (1,)), ((), ())),
                        preferred_element_type=jnp.float32) + bp2_ref[...]
    nrm = jnp.sqrt(jnp.sum(z * z, axis=1, keepdims=True))
    out_ref[...] = z / jnp.clip(nrm, 1e-12, None)


def _pass_post2(outp, hs, dinv, g, b, a, Wp1, bp1, Wp2, bp2):
    return pl.pallas_call(
        _post2_body,
        out_shape=jax.ShapeDtypeStruct((N, 64), jnp.float32),
    )(outp, hs, dinv, g, b, a, Wp1, bp1, Wp2, bp2)


def kernel(X_sp, P_sp, W0, g0, b0, a0, W1, g1, b1, a1, Wp1, bp1, Wp2, bp2):
    Pp = jnp.pad(P_sp, ((0, 0), (0, DIM - 2)))
    ms, mp = _pass_max(X_sp, Pp)
    idx2d, w2d = _pass_topk(X_sp, Pp, ms, mp)
    idxf = idx2d.reshape(-1)
    wf = w2d.reshape(-1)
    idx2dw = jnp.pad(idx2d, ((0, 0), (0, DIM - KP)))
    wgf = _pass_sc_deg(idx2dw, idxf, wf)
    degs = _pass_sc_agg(jnp.ones((N, DIM), jnp.float32), idxf, wf, wgf)

    a0r = a0.reshape(1, 1)
    a1r = a1.reshape(1, 1)
    b0r = b0.reshape(1, DIM)
    g0r = g0.reshape(1, DIM)
    b1r = b1.reshape(1, DIM)
    g1r = g1.reshape(1, DIM)
    bp1r = bp1.reshape(1, DIM)
    bp2r = bp2.reshape(1, 64)

    hs1, dinv = _pass_pre1(X_sp, W0, degs)
    outp1 = _pass_sc_agg(hs1, idxf, wf, wgf)
    h1 = _pass_post1(outp1, hs1, dinv, g0r, b0r, a0r)

    hs2 = _pass_pre2(h1, W1, dinv)
    outp2 = _pass_sc_agg(hs2, idxf, wf, wgf)
    return _pass_post2(outp2, hs2, dinv, g1r, b1r, a1r, Wp1, bp1r, Wp2, bp2r)


# speedup vs baseline: 4.1782x; 1.0816x over previous
import functools

import jax
import jax.numpy as jnp
from jax import lax
from jax.experimental import pallas as pl
from jax.experimental.pallas import tpu as pltpu
from jax.experimental.pallas import tpu_sc as plsc

N = 4096
DIM = 128
KNN = 10
KP = 16
ETA = 0.5
DELTA = 1.0
RB = 256
NBLK = N // RB

NC = 2
NS = 16
NW = NC * NS
RPW = N // NW
CROWS = 8
NCHUNK = RPW // CROWS
CEDGE = CROWS * KP


def _max_body(x_ref, p_ref, ms_ref, mp_ref, acc_ref):
    b = pl.program_id(0)
    x = x_ref[...]
    p = p_ref[...]
    xr = x_ref[pl.ds(b * RB, RB), :]
    pr = p_ref[pl.ds(b * RB, RB), :]

    def d2max(ar, a):
        g = lax.dot_general(ar.astype(jnp.bfloat16), a.astype(jnp.bfloat16),
                            (((1,), (1,)), ((), ())),
                            preferred_element_type=jnp.float32)
        a2r = jnp.sum(ar * ar, axis=1, keepdims=True)
        ones = jnp.ones((1, DIM), jnp.float32)
        a2c = lax.dot_general(ones, a * a, (((1,), (1,)), ((), ())),
                              precision=lax.Precision.HIGHEST,
                              preferred_element_type=jnp.float32)
        return jnp.max(a2r + a2c - 2.0 * g)

    ms = d2max(xr, x)
    mp = d2max(pr, p)

    @pl.when(b == 0)
    def _():
        acc_ref[0] = ms
        acc_ref[1] = mp

    @pl.when(b > 0)
    def _():
        acc_ref[0] = jnp.maximum(acc_ref[0], ms)
        acc_ref[1] = jnp.maximum(acc_ref[1], mp)

    @pl.when(b == NBLK - 1)
    def _():
        ms_ref[0, 0] = jnp.sqrt(jnp.clip(acc_ref[0], 1e-12, None))
        mp_ref[0, 0] = jnp.sqrt(jnp.clip(acc_ref[1], 1e-12, None))


def _pass_max(X, Pp):
    return pl.pallas_call(
        _max_body,
        grid=(NBLK,),
        in_specs=[
            pl.BlockSpec((N, DIM), lambda b: (0, 0)),
            pl.BlockSpec((N, DIM), lambda b: (0, 0)),
        ],
        out_specs=[
            pl.BlockSpec(memory_space=pltpu.SMEM),
            pl.BlockSpec(memory_space=pltpu.SMEM),
        ],
        out_shape=[
            jax.ShapeDtypeStruct((1, 1), jnp.float32),
            jax.ShapeDtypeStruct((1, 1), jnp.float32),
        ],
        scratch_shapes=[pltpu.SMEM((2,), jnp.float32)],
    )(X, Pp)


def _topk_body(x_ref, p_ref, ms_ref, mp_ref, idx_ref, w_ref):
    b = pl.program_id(0)
    x = x_ref[...]
    p = p_ref[...]
    xr = x_ref[pl.ds(b * RB, RB), :]
    pr = p_ref[pl.ds(b * RB, RB), :]
    ms = ms_ref[0, 0]
    mp = mp_ref[0, 0]

    def d2(ar, a):
        g = lax.dot_general(ar.astype(jnp.bfloat16), a.astype(jnp.bfloat16),
                            (((1,), (1,)), ((), ())),
                            preferred_element_type=jnp.float32)
        a2r = jnp.sum(ar * ar, axis=1, keepdims=True)
        ones = jnp.ones((1, DIM), jnp.float32)
        a2c = lax.dot_general(ones, a * a, (((1,), (1,)), ((), ())),
                              precision=lax.Precision.HIGHEST,
                              preferred_element_type=jnp.float32)
        return jnp.clip(a2r + a2c - 2.0 * g, 1e-12, None)

    D = (ETA * (jnp.sqrt(d2(pr, p)) / (mp + 1e-8))
         + (1.0 - ETA) * (jnp.sqrt(d2(xr, x)) / (ms + 1e-8)))

    jglob = lax.broadcasted_iota(jnp.int32, (RB, N), 1)
    ig = lax.broadcasted_iota(jnp.int32, (RB, 1), 0) + b * RB
    Dm = jnp.where(jglob == ig, jnp.inf, D)

    for k in range(KNN):
        m = jnp.min(Dm, axis=1, keepdims=True)
        sel = jnp.min(jnp.where(Dm == m, jglob, N), axis=1,
                      keepdims=True)
        idx_ref[:, k:k + 1] = sel
        w_ref[:, k:k + 1] = jnp.exp(-(m * m) / (DELTA * DELTA + 1e-8))
        Dm = jnp.where(jglob == sel, jnp.inf, Dm)

    for k in range(KNN, KP):
        idx_ref[:, k:k + 1] = ig
        w_ref[:, k:k + 1] = jnp.zeros((RB, 1), jnp.float32)


def _pass_topk(X, Pp, ms, mp):
    return pl.pallas_call(
        _topk_body,
        grid=(NBLK,),
        in_specs=[
            pl.BlockSpec((N, DIM), lambda b: (0, 0)),
            pl.BlockSpec((N, DIM), lambda b: (0, 0)),
            pl.BlockSpec(memory_space=pltpu.SMEM),
            pl.BlockSpec(memory_space=pltpu.SMEM),
        ],
        out_specs=[
            pl.BlockSpec((RB, KP), lambda b: (b, 0)),
            pl.BlockSpec((RB, KP), lambda b: (b, 0)),
        ],
        out_shape=[
            jax.ShapeDtypeStruct((N, KP), jnp.int32),
            jax.ShapeDtypeStruct((N, KP), jnp.float32),
        ],
    )(X, Pp, ms, mp)


def _sc_deg_body(idx2d, idxf, wf, wgf, idx_v, w_v, wg_v, nbr, sem):
    cid = lax.axis_index("c")
    sid = lax.axis_index("s")
    wid = sid * NC + cid
    lane = lax.iota(jnp.int32, 16)
    rots = [((lane + sh) & 15) for sh in (8, 4, 2, 1)]

    def chunk_body(c, _):
        ebase = wid * (RPW * KP) + c * CEDGE
        pltpu.sync_copy(idxf.at[pl.ds(ebase, CEDGE)], idx_v)
        pltpu.sync_copy(wf.at[pl.ds(ebase, CEDGE)], w_v)
        pltpu.async_copy(idx2d.at[idx_v], nbr, sem).wait()

        def grp_body(g, _):
            i_row = wid * RPW + c * CROWS + g
            e0 = g * KP
            wvec = w_v[pl.ds(e0, 16)]
            macc = jnp.zeros((16,), jnp.int32)
            for t in range(KNN):
                row = nbr[e0 + t, pl.ds(0, 16)]
                acc = jnp.where(row == i_row, 1, 0)
                for rot in rots:
                    acc = acc | acc.at[rot].get(mode="promise_in_bounds")
                macc = jnp.where(lane == t, acc, macc)
            wg_v[pl.ds(e0, 16)] = jnp.where(macc > 0, 0.0, wvec)
            return 0

        lax.fori_loop(0, CROWS, grp_body, 0)
        pltpu.sync_copy(wg_v, wgf.at[pl.ds(ebase, CEDGE)])
        return 0

    lax.fori_loop(0, NCHUNK, chunk_body, 0)


def _pass_sc_deg(idx2d, idxf, wf):
    mesh = plsc.VectorSubcoreMesh(core_axis_name="c", subcore_axis_name="s",
                                  num_cores=NC)
    f = functools.partial(
        pl.kernel,
        mesh=mesh,
        out_type=jax.ShapeDtypeStruct((N * KP,), jnp.float32),
        scratch_types=[
            pltpu.VMEM((CEDGE,), jnp.int32),
            pltpu.VMEM((CEDGE,), jnp.float32),
            pltpu.VMEM((CEDGE,), jnp.float32),
            pltpu.VMEM((CEDGE, DIM), jnp.int32),
            pltpu.SemaphoreType.DMA,
        ],
    )(_sc_deg_body)
    return f(idx2d, idxf, wf)


def _sc_agg_body(hs, idxf, wf, wgf, outp, shared, idx_v, w_v, wg_v, gath,
                 valsb, own, gacc, oidx, sem):
    cid = lax.axis_index("c")
    sid = lax.axis_index("s")
    wid = sid * NC + cid
    lane = lax.iota(jnp.int32, 16)
    nvec = DIM // 16

    def zb(i, _):
        gath[i // nvec, pl.ds((i % nvec) * 16, 16)] = jnp.zeros(
            (16,), jnp.float32)
        return 0

    lax.fori_loop(0, CEDGE * nvec, zb, 0)
    pltpu.sync_copy(gath, shared.at[pl.ds(sid * 256, CEDGE)])
    pltpu.sync_copy(gath, shared.at[pl.ds(sid * 256 + CEDGE, CEDGE)])
    plsc.subcore_barrier()

    def gz(i, _):
        gacc[CROWS + i // nvec, pl.ds((i % nvec) * 16, 16)] = jnp.zeros(
            (16,), jnp.float32)
        return 0

    lax.fori_loop(0, (KP - CROWS) * nvec, gz, 0)

    def vz(i, _):
        e = (i // ((KP - KNN) * nvec)) * KP + KNN + (i // nvec) % (KP - KNN)
        gacc_d = (i % nvec) * 16
        valsb[e, pl.ds(gacc_d, 16)] = jnp.zeros((16,), jnp.float32)
        return 0

    lax.fori_loop(0, CROWS * (KP - KNN) * nvec, vz, 0)

    def chunk_body(c, _):
        rowbase = wid * RPW + c * CROWS
        ebase = rowbase * KP
        pltpu.sync_copy(idxf.at[pl.ds(ebase, CEDGE)], idx_v)
        pltpu.sync_copy(wf.at[pl.ds(ebase, CEDGE)], w_v)
        pltpu.sync_copy(wgf.at[pl.ds(ebase, CEDGE)], wg_v)
        pltpu.async_copy(hs.at[idx_v], gath, sem).wait()
        pltpu.sync_copy(hs.at[pl.ds(rowbase, CROWS)], own)

        def row_body(r, _):
            wrow = w_v[pl.ds(r * KP, KP)]
            wgrow = wg_v[pl.ds(r * KP, KP)]
            ownd = [own[r, pl.ds(d * 16, 16)] for d in range(nvec)]
            accs = [jnp.zeros((16,), jnp.float32) for _ in range(nvec)]
            for t in range(KNN):
                wsc = wrow[t]
                wgsc = wgrow[t]
                e = r * KP + t
                for d in range(nvec):
                    valsb[e, pl.ds(d * 16, 16)] = wsc * ownd[d]
                    accs[d] = accs[d] + wgsc * gath[e, pl.ds(d * 16, 16)]
            for d in range(nvec):
                gacc[r, pl.ds(d * 16, 16)] = accs[d]
            return 0

        lax.fori_loop(0, CROWS, row_body, 0)

        oidx[...] = rowbase + (lane & (CROWS - 1))
        pltpu.sync_copy(valsb, shared.at[idx_v], add=True)
        pltpu.sync_copy(gacc, shared.at[oidx], add=True)
        return 0

    lax.fori_loop(0, NCHUNK, chunk_body, 0)
    plsc.subcore_barrier()
    base = sid * 256
    pltpu.sync_copy(shared.at[pl.ds(base, 256)],
                    outp.at[pl.ds(cid * N + base, 256)])


def _pass_sc_agg(hs, idxf, wf, wgf):
    mesh = plsc.VectorSubcoreMesh(core_axis_name="c", subcore_axis_name="s",
                                  num_cores=NC)
    f = functools.partial(
        pl.kernel,
        mesh=mesh,
        out_type=jax.ShapeDtypeStruct((NC * N, DIM), jnp.float32),
        scratch_types=[
            pltpu.VMEM_SHARED((N, DIM), jnp.float32),
            pltpu.VMEM((CEDGE,), jnp.int32),
            pltpu.VMEM((CEDGE,), jnp.float32),
            pltpu.VMEM((CEDGE,), jnp.float32),
            pltpu.VMEM((CEDGE, DIM), jnp.float32),
            pltpu.VMEM((CEDGE, DIM), jnp.float32),
            pltpu.VMEM((CROWS, DIM), jnp.float32),
            pltpu.VMEM((KP, DIM), jnp.float32),
            pltpu.VMEM((16,), jnp.int32),
            pltpu.SemaphoreType.DMA,
        ],
    )(_sc_agg_body)
    return f(hs, idxf, wf, wgf)


def _sc_degsc_body(idxf, wf, degs, shared, idx_v, w_v, valsb, sem):
    cid = lax.axis_index("c")
    sid = lax.axis_index("s")
    wid = sid * NC + cid

    def zb(i, _):
        valsb[i // (DIM // 16), pl.ds((i % (DIM // 16)) * 16, 16)] = (
            jnp.zeros((16,), jnp.float32))
        return 0

    lax.fori_loop(0, CEDGE * (DIM // 16), zb, 0)
    pltpu.sync_copy(valsb, shared.at[pl.ds(sid * 256, CEDGE)])
    pltpu.sync_copy(valsb, shared.at[pl.ds(sid * 256 + CEDGE, CEDGE)])
    plsc.subcore_barrier()

    def chunk_body(c, _):
        ebase = (wid * RPW + c * CROWS) * KP
        pltpu.sync_copy(idxf.at[pl.ds(ebase, CEDGE)], idx_v)
        pltpu.sync_copy(wf.at[pl.ds(ebase, CEDGE)], w_v)

        def grp_body(g, _):
            wvec = w_v[pl.ds(g * KP, 16)]
            for t in range(KNN):
                valsb[g * KP + t, pl.ds(0, 16)] = jnp.full(
                    (16,), wvec[t], jnp.float32)
            return 0

        lax.fori_loop(0, CROWS, grp_body, 0)
        pltpu.sync_copy(valsb, shared.at[idx_v], add=True)
        return 0

    lax.fori_loop(0, NCHUNK, chunk_body, 0)
    plsc.subcore_barrier()
    base = sid * 256
    pltpu.sync_copy(shared.at[pl.ds(base, 256)],
                    degs.at[pl.ds(cid * N + base, 256)])


def _pass_sc_degsc(idxf, wf):
    mesh = plsc.VectorSubcoreMesh(core_axis_name="c", subcore_axis_name="s",
                                  num_cores=NC)
    f = functools.partial(
        pl.kernel,
        mesh=mesh,
        out_type=jax.ShapeDtypeStruct((NC * N, DIM), jnp.float32),
        scratch_types=[
            pltpu.VMEM_SHARED((N, DIM), jnp.float32),
            pltpu.VMEM((CEDGE,), jnp.int32),
            pltpu.VMEM((CEDGE,), jnp.float32),
            pltpu.VMEM((CEDGE, DIM), jnp.float32),
            pltpu.SemaphoreType.DMA,
        ],
    )(_sc_degsc_body)
    return f(idxf, wf)


def _pre1_body(x_ref, w_ref, wg_ref, degs_ref, hs_ref, dinv_ref):
    s_out = jnp.sum(wg_ref[...], axis=1, keepdims=True)
    s_in = degs_ref[pl.ds(0, N), 0:1] + degs_ref[pl.ds(N, N), 0:1]
    deg = jnp.clip(1.0 + s_out + s_in, 1.0, None)
    dinv = deg ** -0.5
    dinv_ref[...] = dinv
    h = lax.dot_general(x_ref[...], w_ref[...], (((1,), (1,)), ((), ())),
                        preferred_element_type=jnp.float32)
    hs_ref[...] = h * dinv


def _pass_pre1(X, W0, wg2d, degs):
    return pl.pallas_call(
        _pre1_body,
        out_shape=[
            jax.ShapeDtypeStruct((N, DIM), jnp.float32),
            jax.ShapeDtypeStruct((N, 1), jnp.float32),
        ],
    )(X, W0, wg2d, degs)


def _pre2_body(x_ref, w_ref, dinv_ref, hs_ref):
    h = lax.dot_general(x_ref[...], w_ref[...], (((1,), (1,)), ((), ())),
                        preferred_element_type=jnp.float32)
    hs_ref[...] = h * dinv_ref[...]


def _pass_pre2(H, W1, dinv):
    return pl.pallas_call(
        _pre2_body,
        out_shape=jax.ShapeDtypeStruct((N, DIM), jnp.float32),
    )(H, W1, dinv)


def _post_math(outp, hs, dinv, g, b, a):
    agg = dinv * (outp[pl.ds(0, N), :] + outp[pl.ds(N, N), :] + hs)
    mu = jnp.mean(agg, axis=0)
    var = jnp.mean((agg - mu) ** 2, axis=0)
    h = (agg - mu) / jnp.sqrt(var + 1e-5) * g + b
    return jnp.where(h > 0, h, a * h)


def _post1_body(outp_ref, hs_ref, dinv_ref, g_ref, b_ref, a_ref, out_ref):
    out_ref[...] = _post_math(outp_ref, hs_ref[...], dinv_ref[...],
                              g_ref[...], b_ref[...], a_ref[...])


def _pass_post1(outp, hs, dinv, g, b, a):
    return pl.pallas_call(
        _post1_body,
        out_shape=jax.ShapeDtypeStruct((N, DIM), jnp.float32),
    )(outp, hs, dinv, g, b, a)


def _post2_body(outp_ref, hs_ref, dinv_ref, g_ref, b_ref, a_ref,
                wp1_ref, bp1_ref, wp2_ref, bp2_ref, out_ref):
    h = _post_math(outp_ref, hs_ref[...], dinv_ref[...],
                   g_ref[...], b_ref[...], a_ref[...])
    z = jnp.maximum(
        lax.dot_general(h, wp1_ref[...], (((1,), (1,)), ((), ())),
                        preferred_element_type=jnp.float32) + bp1_ref[...],
        0.0)
    z = lax.dot_general(z, wp2_ref[...], (((1,), (1,)), ((), ())),
                        preferred_element_type=jnp.float32) + bp2_ref[...]
    nrm = jnp.sqrt(jnp.sum(z * z, axis=1, keepdims=True))
    out_ref[...] = z / jnp.clip(nrm, 1e-12, None)


def _pass_post2(outp, hs, dinv, g, b, a, Wp1, bp1, Wp2, bp2):
    return pl.pallas_call(
        _post2_body,
        out_shape=jax.ShapeDtypeStruct((N, 64), jnp.float32),
    )(outp, hs, dinv, g, b, a, Wp1, bp1, Wp2, bp2)


def kernel(X_sp, P_sp, W0, g0, b0, a0, W1, g1, b1, a1, Wp1, bp1, Wp2, bp2):
    Pp = jnp.pad(P_sp, ((0, 0), (0, DIM - 2)))
    ms, mp = _pass_max(X_sp, Pp)
    idx2d, w2d = _pass_topk(X_sp, Pp, ms, mp)
    idxf = idx2d.reshape(-1)
    wf = w2d.reshape(-1)
    idx2dw = jnp.pad(idx2d, ((0, 0), (0, DIM - KP)))
    wgf = _pass_sc_deg(idx2dw, idxf, wf)
    degs = _pass_sc_degsc(idxf, wf)
    wg2d = wgf.reshape(N, KP)

    a0r = a0.reshape(1, 1)
    a1r = a1.reshape(1, 1)
    b0r = b0.reshape(1, DIM)
    g0r = g0.reshape(1, DIM)
    b1r = b1.reshape(1, DIM)
    g1r = g1.reshape(1, DIM)
    bp1r = bp1.reshape(1, DIM)
    bp2r = bp2.reshape(1, 64)

    hs1, dinv = _pass_pre1(X_sp, W0, wg2d, degs)
    outp1 = _pass_sc_agg(hs1, idxf, wf, wgf)
    h1 = _pass_post1(outp1, hs1, dinv, g0r, b0r, a0r)

    hs2 = _pass_pre2(h1, W1, dinv)
    outp2 = _pass_sc_agg(hs2, idxf, wf, wgf)
    return _pass_post2(outp2, hs2, dinv, g1r, b1r, a1r, Wp1, bp1r, Wp2, bp2r)
